# baseline (speedup 1.0000x reference)
import functools

import jax
import jax.numpy as jnp
from jax import lax
from jax.experimental import pallas as pl
from jax.experimental.pallas import tpu as pltpu
from jax.experimental.pallas import tpu_sc as plsc


def kernel(output, length):
    B, T, D = output.shape
    mesh = plsc.ScalarSubcoreMesh(axis_name="c", num_cores=2)
    half = B // 2

    @functools.partial(
        pl.kernel,
        mesh=mesh,
        out_type=jax.ShapeDtypeStruct((B, D), jnp.float32),
        scratch_types=[
            pltpu.SMEM((B,), jnp.int32),
            pltpu.SemaphoreType.DMA,
        ],
    )
    def gather_last(len_hbm, seq_hbm, out_hbm, len_s, sem):
        cid = lax.axis_index("c")
        base = cid * half
        pltpu.sync_copy(len_hbm, len_s)
        copies = []
        for i in range(half):
            b = base + i
            t = len_s[b] - 1
            copies.append(
                pltpu.make_async_copy(
                    seq_hbm.at[b].at[pl.ds(t, 1)], out_hbm.at[pl.ds(b, 1)], sem
                )
            )
        for c in copies:
            c.start()
        for c in copies:
            c.wait()

    return gather_last(length, output)


# speedup vs baseline: 1.0693x; 1.0693x over previous
import functools

import jax
import jax.numpy as jnp
from jax.experimental import pallas as pl
from jax.experimental.pallas import tpu as pltpu
from jax.experimental.pallas import tpu_sc as plsc


def kernel(output, length):
    B, T, D = output.shape
    mesh = plsc.ScalarSubcoreMesh(axis_name="c", num_cores=1)

    @functools.partial(
        pl.kernel,
        mesh=mesh,
        out_type=jax.ShapeDtypeStruct((B, D), jnp.float32),
        scratch_types=[
            pltpu.SMEM((B,), jnp.int32),
            pltpu.SemaphoreType.DMA,
        ],
    )
    def gather_last(len_hbm, seq_hbm, out_hbm, len_s, sem):
        pltpu.sync_copy(len_hbm, len_s)
        copies = []
        for b in range(B):
            t = len_s[b] - 1
            copies.append(
                pltpu.make_async_copy(
                    seq_hbm.at[b].at[pl.ds(t, 1)], out_hbm.at[pl.ds(b, 1)], sem
                )
            )
        for c in copies:
            c.start()
        for c in copies:
            c.wait()

    return gather_last(length, output)
